# baseline (speedup 1.0000x reference)
import functools
import jax
import jax.numpy as jnp
from jax import lax
from jax.experimental import pallas as pl
from jax.experimental.pallas import tpu as pltpu
from jax.experimental.pallas import tpu_sc as plsc

N = 10000
T = 8
R = 8
B_BASES = 8
H = 256
E = 160000
P = 4096
HH = H // 2

NC = 2
NS = 16
BN = 2000
NB = N // BN

def _mesh():
    return plsc.VectorSubcoreMesh(core_axis_name="c", subcore_axis_name="s")


def _embed_ln_body(ids_ref, ne_ref, te_ref, g_ref, b_ref, out_ref):
    ids = ids_ref[0]
    iota = lax.broadcasted_iota(jnp.int32, (T, 1), 0)
    oh = (iota == ids).astype(jnp.float32)
    emb = lax.dot_general(oh, te_ref[...],
                          dimension_numbers=(((0,), (0,)), ((), ())),
                          preferred_element_type=jnp.float32,
                          precision=lax.Precision.DEFAULT)
    x = ne_ref[...] + emb
    m = jnp.mean(x, axis=-1, keepdims=True)
    v = jnp.mean((x - m) ** 2, axis=-1, keepdims=True)
    out_ref[...] = (x - m) * lax.rsqrt(v + 1e-5) * g_ref[...] + b_ref[...]


def _embed_ln(ids3, node_emb, type_emb, g2, b2):
    return pl.pallas_call(
        _embed_ln_body,
        grid=(NB,),
        in_specs=[
            pl.BlockSpec((1, 1, BN), lambda i: (i, 0, 0)),
            pl.BlockSpec((BN, H), lambda i: (i, 0)),
            pl.BlockSpec((T, H), lambda i: (0, 0)),
            pl.BlockSpec((1, H), lambda i: (0, 0)),
            pl.BlockSpec((1, H), lambda i: (0, 0)),
        ],
        out_specs=pl.BlockSpec((BN, H), lambda i: (i, 0)),
        out_shape=jax.ShapeDtypeStruct((N, H), jnp.float32),
    )(ids3, node_emb, type_emb, g2, b2)


def _wcomp_body(comp_ref, basis_ref, out_ref):
    out_ref[...] = lax.dot_general(
        comp_ref[...], basis_ref[...],
        dimension_numbers=(((1,), (0,)), ((), ())),
        preferred_element_type=jnp.float32,
        precision=lax.Precision.DEFAULT)


def _wcomp(comp, basis_flat):
    CB = 4096
    return pl.pallas_call(
        _wcomp_body,
        grid=(H * H // CB,),
        in_specs=[
            pl.BlockSpec((R, B_BASES), lambda i: (0, 0)),
            pl.BlockSpec((B_BASES, CB), lambda i: (0, i)),
        ],
        out_specs=pl.BlockSpec((R, CB), lambda i: (0, i)),
        out_shape=jax.ShapeDtypeStruct((R, H * H), jnp.float32),
    )(comp, basis_flat)


def _hrel_body(x_ref, w_ref, lo_ref, hi_ref):
    f = lax.dot_general(x_ref[...], w_ref[0],
                        dimension_numbers=(((1,), (0,)), ((), ())),
                        preferred_element_type=jnp.float32,
                        precision=lax.Precision.DEFAULT)
    lo_ref[0] = f[:, :HH]
    hi_ref[0] = f[:, HH:]


def _hrel(x, wstack):
    return pl.pallas_call(
        _hrel_body,
        grid=(R, NB),
        in_specs=[
            pl.BlockSpec((BN, H), lambda r, i: (i, 0)),
            pl.BlockSpec((1, H, H), lambda r, i: (r, 0, 0)),
        ],
        out_specs=[
            pl.BlockSpec((1, BN, HH), lambda r, i: (r, i, 0)),
            pl.BlockSpec((1, BN, HH), lambda r, i: (r, i, 0)),
        ],
        out_shape=[
            jax.ShapeDtypeStruct((R, N, HH), jnp.float32),
            jax.ShapeDtypeStruct((R, N, HH), jnp.float32),
        ],
    )(x, wstack)


def _update_body(x_ref, agg_ref, root_ref, bias_ref, g_ref, b_ref, out_ref):
    x = x_ref[...]
    agg = jnp.concatenate([agg_ref[0], agg_ref[1]], axis=-1)
    upd = agg + lax.dot_general(x, root_ref[...],
                                dimension_numbers=(((1,), (0,)), ((), ())),
                                preferred_element_type=jnp.float32,
                                precision=lax.Precision.DEFAULT) + bias_ref[...]
    t = x + 0.5 * upd * (1.0 + lax.erf(upd * 0.7071067811865476))
    m = jnp.mean(t, axis=-1, keepdims=True)
    v = jnp.mean((t - m) ** 2, axis=-1, keepdims=True)
    out_ref[...] = (t - m) * lax.rsqrt(v + 1e-5) * g_ref[...] + b_ref[...]


def _update(x, agg2, root, bias2, g2, b2):
    return pl.pallas_call(
        _update_body,
        grid=(NB,),
        in_specs=[
            pl.BlockSpec((BN, H), lambda i: (i, 0)),
            pl.BlockSpec((2, BN, HH), lambda i: (0, i, 0)),
            pl.BlockSpec((H, H), lambda i: (0, 0)),
            pl.BlockSpec((1, H), lambda i: (0, 0)),
            pl.BlockSpec((1, H), lambda i: (0, 0)),
            pl.BlockSpec((1, H), lambda i: (0, 0)),
        ],
        out_specs=pl.BlockSpec((BN, H), lambda i: (i, 0)),
        out_shape=jax.ShapeDtypeStruct((N, H), jnp.float32),
    )(x, agg2, root, bias2, g2, b2)


def _mlp_body(d_ref, s_ref, pw1_ref, pb1_ref, pw2_ref, pb2_ref, pw3_ref,
              pb3_ref, out_ref):
    d = d_ref[...]
    s = s_ref[...]
    dn = lax.Precision.DEFAULT

    def mm(a, w):
        return lax.dot_general(a, w, dimension_numbers=(((1,), (0,)), ((), ())),
                               preferred_element_type=jnp.float32, precision=dn)

    acc = mm(d, pw1_ref[pl.ds(0, H), :])
    acc += mm(s, pw1_ref[pl.ds(H, H), :])
    acc += mm(jnp.abs(d - s), pw1_ref[pl.ds(2 * H, H), :])
    acc += mm(d * s, pw1_ref[pl.ds(3 * H, H), :])
    h1 = acc + pb1_ref[...]
    h1 = 0.5 * h1 * (1.0 + lax.erf(h1 * 0.7071067811865476))
    h2 = mm(h1, pw2_ref[...]) + pb2_ref[...]
    h2 = 0.5 * h2 * (1.0 + lax.erf(h2 * 0.7071067811865476))
    out_ref[...] = mm(h2, pw3_ref[...]) + pb3_ref[...]


def _mlp(d, s, pw1, pb1_2, pw2, pb2_2, pw3, pb3_2):
    BP = 1024
    return pl.pallas_call(
        _mlp_body,
        grid=(P // BP,),
        in_specs=[
            pl.BlockSpec((BP, H), lambda i: (i, 0)),
            pl.BlockSpec((BP, H), lambda i: (i, 0)),
            pl.BlockSpec((4 * H, H), lambda i: (0, 0)),
            pl.BlockSpec((1, H), lambda i: (0, 0)),
            pl.BlockSpec((H, H), lambda i: (0, 0)),
            pl.BlockSpec((1, H), lambda i: (0, 0)),
            pl.BlockSpec((H, 1), lambda i: (0, 0)),
            pl.BlockSpec((1, 1), lambda i: (0, 0)),
        ],
        out_specs=pl.BlockSpec((BP, 1), lambda i: (i, 0)),
        out_shape=jax.ShapeDtypeStruct((P, 1), jnp.float32),
    )(d, s, pw1, pb1_2, pw2, pb2_2, pw3, pb3_2)


C1 = 2000
EPT = E // NS
NR = N * R
CPT = NR // NS


def _prep_body(src_hbm, dst_hbm, typ_hbm, w_hbm, gidx_hbm,
               cnt_sh, cnt_loc, src_c, dst_c, typ_c, seg_c, gidx_c, w_c,
               ones_c):
    c = lax.axis_index("c")
    sid = lax.axis_index("s")

    @pl.when(c == 0)
    def _():
        def f0(i, _):
            ones_c[pl.ds(i * 16, 16)] = jnp.full((16,), 1.0, jnp.float32)
            return 0
        lax.fori_loop(0, C1 // 16, f0, 0)

        def f1(i, _):
            cnt_loc[pl.ds(i * 16, 16)] = jnp.zeros((16,), jnp.float32)
            return 0
        lax.fori_loop(0, (CPT + 15) // 16, f1, 0)
        pltpu.sync_copy(cnt_loc.at[pl.ds(0, CPT)],
                        cnt_sh.at[pl.ds(sid * CPT, CPT)])
        plsc.subcore_barrier()

        def chunk_a(k, _):
            base = sid * EPT + k * C1
            pltpu.sync_copy(src_hbm.at[pl.ds(base, C1)], src_c)
            pltpu.sync_copy(dst_hbm.at[pl.ds(base, C1)], dst_c)
            pltpu.sync_copy(typ_hbm.at[pl.ds(base, C1)], typ_c)

            def vec(j, _):
                sl = pl.ds(j * 16, 16)
                dv = dst_c[sl]
                tv = typ_c[sl]
                seg_c[sl] = dv * R + tv
                gidx_c[sl] = tv * N + src_c[sl]
                return 0
            lax.fori_loop(0, C1 // 16, vec, 0)
            pltpu.sync_copy(ones_c, cnt_sh.at[seg_c], add=True)
            pltpu.sync_copy(gidx_c, gidx_hbm.at[pl.ds(base, C1)])
            return 0
        lax.fori_loop(0, EPT // C1, chunk_a, 0)
        plsc.subcore_barrier()

        pltpu.sync_copy(cnt_sh, cnt_loc)

        def chunk_b(k, _):
            base = sid * EPT + k * C1
            pltpu.sync_copy(dst_hbm.at[pl.ds(base, C1)], dst_c)
            pltpu.sync_copy(typ_hbm.at[pl.ds(base, C1)], typ_c)

            def vec(j, _):
                sl = pl.ds(j * 16, 16)
                sv = dst_c[sl] * R + typ_c[sl]
                cv = plsc.load_gather(cnt_loc, [sv])
                w_c[sl] = 1.0 / jnp.maximum(cv, 1.0)
                return 0
            lax.fori_loop(0, C1 // 16, vec, 0)
            pltpu.sync_copy(w_c, w_hbm.at[pl.ds(base, C1)])
            return 0
        lax.fori_loop(0, EPT // C1, chunk_b, 0)


def _sc_prep(edge_src, edge_dst, edge_type):
    f = pl.kernel(
        _prep_body,
        mesh=_mesh(),
        compiler_params=pltpu.CompilerParams(needs_layout_passes=False),
        out_type=[
            jax.ShapeDtypeStruct((E,), jnp.float32),
            jax.ShapeDtypeStruct((E,), jnp.int32),
        ],
        scratch_types=[
            pltpu.VMEM_SHARED((NR,), jnp.float32),
            pltpu.VMEM((NR,), jnp.float32),
            pltpu.VMEM((C1,), jnp.int32),
            pltpu.VMEM((C1,), jnp.int32),
            pltpu.VMEM((C1,), jnp.int32),
            pltpu.VMEM((C1,), jnp.int32),
            pltpu.VMEM((C1,), jnp.int32),
            pltpu.VMEM((C1,), jnp.float32),
            pltpu.VMEM((C1,), jnp.float32),
        ],
    )
    return f(edge_src, edge_dst, edge_type)


C2 = 104
EPT2 = 10400
NCH = EPT2 // C2
EPAD = EPT2 * NS
NPAD = N + 8
RPT = 624


def _edge_body(hlo_hbm, hhi_hbm, gidx_hbm, dst_hbm, w_hbm, out_hbm,
               agg_sh, rows0, rows1,
               gidx0, gidx1, gidx2, gidx3,
               dst0, dst1, dst2, dst3,
               w0, w1, w2, w3,
               semg0, semg1, semm0, semm1, semm2, semm3):
    c = lax.axis_index("c")
    sid = lax.axis_index("s")
    base_r = sid * RPT
    base_e = sid * EPT2
    rows = (rows0, rows1)
    gidxs = (gidx0, gidx1, gidx2, gidx3)
    dsts = (dst0, dst1, dst2, dst3)
    ws = (w0, w1, w2, w3)
    semg = (semg0, semg1)
    semm = (semm0, semm1, semm2, semm3)

    def issue_meta(k, m, sem):
        base = base_e + k * C2
        pltpu.async_copy(gidx_hbm.at[pl.ds(base, C2)], gidxs[m], sem)
        pltpu.async_copy(dst_hbm.at[pl.ds(base, C2)], dsts[m], sem)
        pltpu.async_copy(w_hbm.at[pl.ds(base, C2)], ws[m], sem)

    def wait_meta(m, sem):
        pltpu.make_async_copy(gidx_hbm.at[pl.ds(0, C2)], gidxs[m], sem).wait()
        pltpu.make_async_copy(dst_hbm.at[pl.ds(0, C2)], dsts[m], sem).wait()
        pltpu.make_async_copy(w_hbm.at[pl.ds(0, C2)], ws[m], sem).wait()

    def issue_gather(m, rb, sem):
        @pl.when(c == 0)
        def _():
            pltpu.async_copy(hlo_hbm.at[gidxs[m]], rows[rb], sem)

        @pl.when(c == 1)
        def _():
            pltpu.async_copy(hhi_hbm.at[gidxs[m]], rows[rb], sem)

    def wait_gather(rb, sem):
        pltpu.make_async_copy(hlo_hbm.at[gidxs[0]], rows[rb], sem).wait()

    def z0(e, _):
        for j in range(HH // 16):
            rows0[e, pl.ds(j * 16, 16)] = jnp.zeros((16,), jnp.float32)
        return 0
    lax.fori_loop(0, C2, z0, 0)

    def zc(i, _):
        pltpu.sync_copy(rows0, agg_sh.at[pl.ds(base_r + i * C2, C2)])
        return 0
    lax.fori_loop(0, RPT // C2, zc, 0)

    @pl.when(sid == NS - 1)
    def _():
        pltpu.sync_copy(rows0.at[pl.ds(0, N - NS * RPT)],
                        agg_sh.at[pl.ds(NS * RPT, N - NS * RPT)])
    plsc.subcore_barrier()

    for m in range(4):
        issue_meta(m, m, semm[m])
    wait_meta(0, semm[0])
    issue_gather(0, 0, semg[0])
    wait_meta(1, semm[1])
    issue_gather(1, 1, semg[1])

    def quad(k4, _):
        for i in range(4):
            jb = 4 * k4 + i
            rb = i % 2
            mb = i % 4
            wait_gather(rb, semg[rb])
            wv_ref = ws[mb]

            def scale(e, _):
                wv = plsc.load_gather(wv_ref,
                                      [jnp.full((16,), e, jnp.int32)])
                for j in range(HH // 16):
                    sl = pl.ds(j * 16, 16)
                    rows[rb][e, sl] = rows[rb][e, sl] * wv
                return 0
            lax.fori_loop(0, C2, scale, 0, unroll=2)
            pltpu.sync_copy(rows[rb], agg_sh.at[dsts[mb]], add=True)

            @pl.when(jb + 4 < NCH)
            def _():
                issue_meta(jb + 4, mb, semm[mb])

            @pl.when(jb + 2 < NCH)
            def _():
                wait_meta((mb + 2) % 4, semm[(mb + 2) % 4])
                issue_gather((mb + 2) % 4, rb, semg[rb])
        return 0
    lax.fori_loop(0, NCH // 4, quad, 0)

    plsc.subcore_barrier()
    pltpu.sync_copy(agg_sh.at[pl.ds(base_r, RPT)],
                    out_hbm.at[pl.ds(c * N + base_r, RPT)])

    @pl.when(sid == NS - 1)
    def _():
        pltpu.sync_copy(agg_sh.at[pl.ds(NS * RPT, N - NS * RPT)],
                        out_hbm.at[pl.ds(c * N + NS * RPT, N - NS * RPT)])


def _sc_edge(hlo, hhi, gidx, dst, w):
    f = pl.kernel(
        _edge_body,
        mesh=_mesh(),
        compiler_params=pltpu.CompilerParams(needs_layout_passes=False),
        out_type=jax.ShapeDtypeStruct((2 * N, HH), jnp.float32),
        scratch_types=(
            [pltpu.VMEM_SHARED((NPAD, HH), jnp.float32)]
            + [pltpu.VMEM((C2, HH), jnp.float32)] * 2
            + [pltpu.VMEM((C2,), jnp.int32)] * 8
            + [pltpu.VMEM((C2,), jnp.float32)] * 4
            + [pltpu.SemaphoreType.DMA] * 6
        ),
    )
    return f(hlo, hhi, gidx, dst, w)


PG = 2 * P
GPW = PG // (NC * NS)


def _pair_body(hid_hbm, idx_hbm, out_hbm, idx_c, rows, sem):
    c = lax.axis_index("c")
    sid = lax.axis_index("s")
    wid = sid * NC + c
    base = wid * GPW
    pltpu.sync_copy(idx_hbm.at[pl.ds(base, GPW)], idx_c)
    pltpu.async_copy(hid_hbm.at[idx_c], rows, sem).wait()
    pltpu.sync_copy(rows, out_hbm.at[pl.ds(base, GPW)])


def _pair_gather(hidden, all_idx):
    f = pl.kernel(
        _pair_body,
        mesh=_mesh(),
        compiler_params=pltpu.CompilerParams(needs_layout_passes=False),
        out_type=jax.ShapeDtypeStruct((PG, H), jnp.float32),
        scratch_types=[
            pltpu.VMEM((GPW,), jnp.int32),
            pltpu.VMEM((GPW, H), jnp.float32),
            pltpu.SemaphoreType.DMA,
        ],
    )
    return f(hidden, all_idx)


def kernel(node_emb, type_emb, ln_in_g, ln_in_b,
           basis0, comp0, root0, bias0, ln0_g, ln0_b,
           basis1, comp1, root1, bias1, ln1_g, ln1_b,
           pw1, pb1, pw2, pb2, pw3, pb3,
           node_type_ids, edge_index, edge_type, drug_indices,
           disease_indices):
    edge_src = edge_index[0]
    edge_dst = edge_index[1]
    r2 = lambda a: a.reshape(1, -1)

    w, gidx = _sc_prep(edge_src, edge_dst, edge_type)

    def pad2(a, fill):
        return jnp.pad(a.reshape(NS, E // NS), ((0, 0), (0, EPT2 - E // NS)),
                       constant_values=fill).reshape(EPAD)
    w_p = pad2(w, 0.0)
    gidx_p = pad2(gidx, 0)
    dst_p = pad2(edge_dst, N)

    ids3 = node_type_ids.reshape(NB, 1, BN)
    hidden = _embed_ln(ids3, node_emb, type_emb, r2(ln_in_g), r2(ln_in_b))

    for basis, comp, root, bias, g, b in (
            (basis0, comp0, root0, bias0, ln0_g, ln0_b),
            (basis1, comp1, root1, bias1, ln1_g, ln1_b)):
        wflat = _wcomp(comp, basis.reshape(B_BASES, H * H))
        wstack = wflat.reshape(R, H, H)
        hlo, hhi = _hrel(hidden, wstack)
        agg2 = _sc_edge(hlo.reshape(R * N, HH), hhi.reshape(R * N, HH),
                        gidx_p, dst_p, w_p).reshape(2, N, HH)
        hidden = _update(hidden, agg2, root, r2(bias), r2(g), r2(b))

    all_idx = jnp.concatenate([drug_indices, disease_indices])
    pairs = _pair_gather(hidden, all_idx)
    logits = _mlp(pairs[:P], pairs[P:], pw1, r2(pb1), pw2, r2(pb2),
                  pw3, pb3.reshape(1, 1))
    return logits.reshape(P)


# speedup vs baseline: 1.2298x; 1.2298x over previous
import functools
import jax
import jax.numpy as jnp
from jax import lax
from jax.experimental import pallas as pl
from jax.experimental.pallas import tpu as pltpu
from jax.experimental.pallas import tpu_sc as plsc

N = 10000
T = 8
R = 8
B_BASES = 8
H = 256
E = 160000
P = 4096
HH = H // 2

NC = 2
NS = 16
BN = 2000
NB = N // BN

def _mesh():
    return plsc.VectorSubcoreMesh(core_axis_name="c", subcore_axis_name="s")


def _embed_ln_body(ids_ref, ne_ref, te_ref, g_ref, b_ref, out_ref):
    ids = ids_ref[0]
    iota = lax.broadcasted_iota(jnp.int32, (T, 1), 0)
    oh = (iota == ids).astype(jnp.float32)
    emb = lax.dot_general(oh, te_ref[...],
                          dimension_numbers=(((0,), (0,)), ((), ())),
                          preferred_element_type=jnp.float32,
                          precision=lax.Precision.DEFAULT)
    x = ne_ref[...] + emb
    m = jnp.mean(x, axis=-1, keepdims=True)
    v = jnp.mean((x - m) ** 2, axis=-1, keepdims=True)
    out_ref[...] = (x - m) * lax.rsqrt(v + 1e-5) * g_ref[...] + b_ref[...]


def _embed_ln(ids3, node_emb, type_emb, g2, b2):
    return pl.pallas_call(
        _embed_ln_body,
        grid=(NB,),
        in_specs=[
            pl.BlockSpec((1, 1, BN), lambda i: (i, 0, 0)),
            pl.BlockSpec((BN, H), lambda i: (i, 0)),
            pl.BlockSpec((T, H), lambda i: (0, 0)),
            pl.BlockSpec((1, H), lambda i: (0, 0)),
            pl.BlockSpec((1, H), lambda i: (0, 0)),
        ],
        out_specs=pl.BlockSpec((BN, H), lambda i: (i, 0)),
        out_shape=jax.ShapeDtypeStruct((N, H), jnp.float32),
    )(ids3, node_emb, type_emb, g2, b2)


def _wcomp_body(comp_ref, basis_ref, out_ref):
    out_ref[...] = lax.dot_general(
        comp_ref[...], basis_ref[...],
        dimension_numbers=(((1,), (0,)), ((), ())),
        preferred_element_type=jnp.float32,
        precision=lax.Precision.DEFAULT)


def _wcomp(comp, basis_flat):
    CB = 4096
    return pl.pallas_call(
        _wcomp_body,
        grid=(H * H // CB,),
        in_specs=[
            pl.BlockSpec((R, B_BASES), lambda i: (0, 0)),
            pl.BlockSpec((B_BASES, CB), lambda i: (0, i)),
        ],
        out_specs=pl.BlockSpec((R, CB), lambda i: (0, i)),
        out_shape=jax.ShapeDtypeStruct((R, H * H), jnp.float32),
    )(comp, basis_flat)


def _hrel_body(x_ref, w_ref, lo_ref, hi_ref):
    f = lax.dot_general(x_ref[...], w_ref[0],
                        dimension_numbers=(((1,), (0,)), ((), ())),
                        preferred_element_type=jnp.float32,
                        precision=lax.Precision.DEFAULT)
    lo_ref[0] = f[:, :HH]
    hi_ref[0] = f[:, HH:]


def _hrel(x, wstack):
    return pl.pallas_call(
        _hrel_body,
        grid=(R, NB),
        in_specs=[
            pl.BlockSpec((BN, H), lambda r, i: (i, 0)),
            pl.BlockSpec((1, H, H), lambda r, i: (r, 0, 0)),
        ],
        out_specs=[
            pl.BlockSpec((1, BN, HH), lambda r, i: (r, i, 0)),
            pl.BlockSpec((1, BN, HH), lambda r, i: (r, i, 0)),
        ],
        out_shape=[
            jax.ShapeDtypeStruct((R, N, HH), jnp.float32),
            jax.ShapeDtypeStruct((R, N, HH), jnp.float32),
        ],
    )(x, wstack)


def _update_body(x_ref, agg_ref, root_ref, bias_ref, g_ref, b_ref, out_ref):
    x = x_ref[...]
    agg = jnp.concatenate([agg_ref[0], agg_ref[1]], axis=-1)
    upd = agg + lax.dot_general(x, root_ref[...],
                                dimension_numbers=(((1,), (0,)), ((), ())),
                                preferred_element_type=jnp.float32,
                                precision=lax.Precision.DEFAULT) + bias_ref[...]
    t = x + 0.5 * upd * (1.0 + lax.erf(upd * 0.7071067811865476))
    m = jnp.mean(t, axis=-1, keepdims=True)
    v = jnp.mean((t - m) ** 2, axis=-1, keepdims=True)
    out_ref[...] = (t - m) * lax.rsqrt(v + 1e-5) * g_ref[...] + b_ref[...]


def _update(x, agg2, root, bias2, g2, b2):
    return pl.pallas_call(
        _update_body,
        grid=(NB,),
        in_specs=[
            pl.BlockSpec((BN, H), lambda i: (i, 0)),
            pl.BlockSpec((2, BN, HH), lambda i: (0, i, 0)),
            pl.BlockSpec((H, H), lambda i: (0, 0)),
            pl.BlockSpec((1, H), lambda i: (0, 0)),
            pl.BlockSpec((1, H), lambda i: (0, 0)),
            pl.BlockSpec((1, H), lambda i: (0, 0)),
        ],
        out_specs=pl.BlockSpec((BN, H), lambda i: (i, 0)),
        out_shape=jax.ShapeDtypeStruct((N, H), jnp.float32),
    )(x, agg2, root, bias2, g2, b2)


def _mlp_body(d_ref, s_ref, pw1_ref, pb1_ref, pw2_ref, pb2_ref, pw3_ref,
              pb3_ref, out_ref):
    d = d_ref[...]
    s = s_ref[...]
    dn = lax.Precision.DEFAULT

    def mm(a, w):
        return lax.dot_general(a, w, dimension_numbers=(((1,), (0,)), ((), ())),
                               preferred_element_type=jnp.float32, precision=dn)

    acc = mm(d, pw1_ref[pl.ds(0, H), :])
    acc += mm(s, pw1_ref[pl.ds(H, H), :])
    acc += mm(jnp.abs(d - s), pw1_ref[pl.ds(2 * H, H), :])
    acc += mm(d * s, pw1_ref[pl.ds(3 * H, H), :])
    h1 = acc + pb1_ref[...]
    h1 = 0.5 * h1 * (1.0 + lax.erf(h1 * 0.7071067811865476))
    h2 = mm(h1, pw2_ref[...]) + pb2_ref[...]
    h2 = 0.5 * h2 * (1.0 + lax.erf(h2 * 0.7071067811865476))
    out_ref[...] = mm(h2, pw3_ref[...]) + pb3_ref[...]


def _mlp(d, s, pw1, pb1_2, pw2, pb2_2, pw3, pb3_2):
    BP = 1024
    return pl.pallas_call(
        _mlp_body,
        grid=(P // BP,),
        in_specs=[
            pl.BlockSpec((BP, H), lambda i: (i, 0)),
            pl.BlockSpec((BP, H), lambda i: (i, 0)),
            pl.BlockSpec((4 * H, H), lambda i: (0, 0)),
            pl.BlockSpec((1, H), lambda i: (0, 0)),
            pl.BlockSpec((H, H), lambda i: (0, 0)),
            pl.BlockSpec((1, H), lambda i: (0, 0)),
            pl.BlockSpec((H, 1), lambda i: (0, 0)),
            pl.BlockSpec((1, 1), lambda i: (0, 0)),
        ],
        out_specs=pl.BlockSpec((BP, 1), lambda i: (i, 0)),
        out_shape=jax.ShapeDtypeStruct((P, 1), jnp.float32),
    )(d, s, pw1, pb1_2, pw2, pb2_2, pw3, pb3_2)


C1 = 2000
EPT = E // NS
NR = N * R
CPT = NR // NS


def _prep_body(src_hbm, dst_hbm, typ_hbm, w_hbm, gidx_hbm,
               cnt_sh, cnt_loc, src_c, dst_c, typ_c, seg_c, gidx_c, w_c,
               ones_c):
    c = lax.axis_index("c")
    sid = lax.axis_index("s")

    @pl.when(c == 0)
    def _():
        def f0(i, _):
            ones_c[pl.ds(i * 16, 16)] = jnp.full((16,), 1.0, jnp.float32)
            return 0
        lax.fori_loop(0, C1 // 16, f0, 0)

        def f1(i, _):
            cnt_loc[pl.ds(i * 16, 16)] = jnp.zeros((16,), jnp.float32)
            return 0
        lax.fori_loop(0, (CPT + 15) // 16, f1, 0)
        pltpu.sync_copy(cnt_loc.at[pl.ds(0, CPT)],
                        cnt_sh.at[pl.ds(sid * CPT, CPT)])
        plsc.subcore_barrier()

        def chunk_a(k, _):
            base = sid * EPT + k * C1
            pltpu.sync_copy(src_hbm.at[pl.ds(base, C1)], src_c)
            pltpu.sync_copy(dst_hbm.at[pl.ds(base, C1)], dst_c)
            pltpu.sync_copy(typ_hbm.at[pl.ds(base, C1)], typ_c)

            def vec(j, _):
                sl = pl.ds(j * 16, 16)
                dv = dst_c[sl]
                tv = typ_c[sl]
                seg_c[sl] = dv * R + tv
                gidx_c[sl] = tv * N + src_c[sl]
                return 0
            lax.fori_loop(0, C1 // 16, vec, 0)
            pltpu.sync_copy(ones_c, cnt_sh.at[seg_c], add=True)
            pltpu.sync_copy(gidx_c, gidx_hbm.at[pl.ds(base, C1)])
            return 0
        lax.fori_loop(0, EPT // C1, chunk_a, 0)
        plsc.subcore_barrier()

        pltpu.sync_copy(cnt_sh, cnt_loc)

        def chunk_b(k, _):
            base = sid * EPT + k * C1
            pltpu.sync_copy(dst_hbm.at[pl.ds(base, C1)], dst_c)
            pltpu.sync_copy(typ_hbm.at[pl.ds(base, C1)], typ_c)

            def vec(j, _):
                sl = pl.ds(j * 16, 16)
                sv = dst_c[sl] * R + typ_c[sl]
                cv = plsc.load_gather(cnt_loc, [sv])
                w_c[sl] = 1.0 / jnp.maximum(cv, 1.0)
                return 0
            lax.fori_loop(0, C1 // 16, vec, 0)
            pltpu.sync_copy(w_c, w_hbm.at[pl.ds(base, C1)])
            return 0
        lax.fori_loop(0, EPT // C1, chunk_b, 0)


def _sc_prep(edge_src, edge_dst, edge_type):
    f = pl.kernel(
        _prep_body,
        mesh=_mesh(),
        compiler_params=pltpu.CompilerParams(needs_layout_passes=False),
        out_type=[
            jax.ShapeDtypeStruct((E,), jnp.float32),
            jax.ShapeDtypeStruct((E,), jnp.int32),
        ],
        scratch_types=[
            pltpu.VMEM_SHARED((NR,), jnp.float32),
            pltpu.VMEM((NR,), jnp.float32),
            pltpu.VMEM((C1,), jnp.int32),
            pltpu.VMEM((C1,), jnp.int32),
            pltpu.VMEM((C1,), jnp.int32),
            pltpu.VMEM((C1,), jnp.int32),
            pltpu.VMEM((C1,), jnp.int32),
            pltpu.VMEM((C1,), jnp.float32),
            pltpu.VMEM((C1,), jnp.float32),
        ],
    )
    return f(edge_src, edge_dst, edge_type)


C2 = 128
CPB = 8
BE = C2 * CPB
NCH = 80
NMB = NCH // CPB
EPT2 = C2 * NCH
EPAD = EPT2 * NS
MROWS = EPAD // C2
NPAD = N + 8
RPT = 624


def _edge_body(hlo_hbm, hhi_hbm, gidx_hbm, dst_hbm, w_hbm, out_hbm,
               agg_sh, rows0, rows1, gidx0, gidx1, dst0, dst1, w0, w1,
               semg0, semg1, semm0, semm1):
    c = lax.axis_index("c")
    sid = lax.axis_index("s")
    base_r = sid * RPT
    mrow0 = sid * NCH
    rows = (rows0, rows1)
    gidxs = (gidx0, gidx1)
    dsts = (dst0, dst1)
    ws = (w0, w1)
    semg = (semg0, semg1)
    semm = (semm0, semm1)

    def issue_meta(bi, m):
        r0 = mrow0 + bi * CPB
        pltpu.async_copy(gidx_hbm.at[pl.ds(r0, CPB)], gidxs[m], semm[m])
        pltpu.async_copy(dst_hbm.at[pl.ds(r0, CPB)], dsts[m], semm[m])
        pltpu.async_copy(w_hbm.at[pl.ds(r0, CPB)], ws[m], semm[m])

    def wait_meta(m):
        pltpu.make_async_copy(gidx_hbm.at[pl.ds(0, CPB)], gidxs[m],
                              semm[m]).wait()
        pltpu.make_async_copy(dst_hbm.at[pl.ds(0, CPB)], dsts[m],
                              semm[m]).wait()
        pltpu.make_async_copy(w_hbm.at[pl.ds(0, CPB)], ws[m],
                              semm[m]).wait()

    def issue_gather(m, i, rb):
        idx = gidxs[m].at[i]

        @pl.when(c == 0)
        def _():
            pltpu.async_copy(hlo_hbm.at[idx], rows[rb], semg[rb])

        @pl.when(c == 1)
        def _():
            pltpu.async_copy(hhi_hbm.at[idx], rows[rb], semg[rb])

    def wait_gather(rb):
        pltpu.make_async_copy(hlo_hbm.at[gidxs[0].at[0]], rows[rb],
                              semg[rb]).wait()

    def z0(e, _):
        for j in range(HH // 16):
            rows0[e, pl.ds(j * 16, 16)] = jnp.zeros((16,), jnp.float32)
        return 0
    lax.fori_loop(0, C2, z0, 0)

    def zc(i, _):
        pltpu.sync_copy(rows0, agg_sh.at[pl.ds(base_r + i * C2, C2)])
        return 0
    lax.fori_loop(0, RPT // C2, zc, 0)
    pltpu.sync_copy(rows0.at[pl.ds(0, RPT % C2)],
                    agg_sh.at[pl.ds(base_r + (RPT // C2) * C2, RPT % C2)])

    @pl.when(sid == NS - 1)
    def _():
        pltpu.sync_copy(rows0.at[pl.ds(0, N - NS * RPT)],
                        agg_sh.at[pl.ds(NS * RPT, N - NS * RPT)])
    plsc.subcore_barrier()

    issue_meta(0, 0)
    wait_meta(0)
    issue_gather(0, 0, 0)

    def pair(bp, _):
        for sb in range(2):
            bi = 2 * bp + sb

            @pl.when(bi + 1 < NMB)
            def _():
                issue_meta(bi + 1, 1 - sb)
            for i in range(CPB):
                rb = i % 2

                if i == CPB - 1:
                    @pl.when(bi + 1 < NMB)
                    def _():
                        wait_meta(1 - sb)
                        issue_gather(1 - sb, 0, 1 - rb)
                else:
                    issue_gather(sb, i + 1, 1 - rb)
                wait_gather(rb)
                wref = ws[sb]
                dref = dsts[sb]
                rref = rows[rb]

                def scale(e, _):
                    wv = plsc.load_gather(
                        wref, [jnp.full((16,), i, jnp.int32),
                               jnp.full((16,), e, jnp.int32)])
                    for jj in range(HH // 16):
                        sl = pl.ds(jj * 16, 16)
                        rref[e, sl] = rref[e, sl] * wv
                    return 0
                lax.fori_loop(0, C2, scale, 0, unroll=2)
                pltpu.sync_copy(rref, agg_sh.at[dref.at[i]], add=True)
        return 0
    lax.fori_loop(0, NMB // 2, pair, 0)

    plsc.subcore_barrier()
    pltpu.sync_copy(agg_sh.at[pl.ds(base_r, RPT)],
                    out_hbm.at[pl.ds(c * N + base_r, RPT)])

    @pl.when(sid == NS - 1)
    def _():
        pltpu.sync_copy(agg_sh.at[pl.ds(NS * RPT, N - NS * RPT)],
                        out_hbm.at[pl.ds(c * N + NS * RPT, N - NS * RPT)])


def _sc_edge(hlo, hhi, gidx, dst, w):
    f = pl.kernel(
        _edge_body,
        mesh=_mesh(),
        compiler_params=pltpu.CompilerParams(needs_layout_passes=False),
        out_type=jax.ShapeDtypeStruct((2 * N, HH), jnp.float32),
        scratch_types=(
            [pltpu.VMEM_SHARED((NPAD, HH), jnp.float32)]
            + [pltpu.VMEM((C2, HH), jnp.float32)] * 2
            + [pltpu.VMEM((CPB, C2), jnp.int32)] * 4
            + [pltpu.VMEM((CPB, C2), jnp.float32)] * 2
            + [pltpu.SemaphoreType.DMA] * 4
        ),
    )
    return f(hlo, hhi, gidx, dst, w)


PG = 2 * P
GPW = PG // (NC * NS)


def _pair_body(hid_hbm, idx_hbm, out_hbm, idx_c, rows, sem):
    c = lax.axis_index("c")
    sid = lax.axis_index("s")
    wid = sid * NC + c
    base = wid * GPW
    pltpu.sync_copy(idx_hbm.at[pl.ds(base, GPW)], idx_c)
    pltpu.async_copy(hid_hbm.at[idx_c], rows, sem).wait()
    pltpu.sync_copy(rows, out_hbm.at[pl.ds(base, GPW)])


def _pair_gather(hidden, all_idx):
    f = pl.kernel(
        _pair_body,
        mesh=_mesh(),
        compiler_params=pltpu.CompilerParams(needs_layout_passes=False),
        out_type=jax.ShapeDtypeStruct((PG, H), jnp.float32),
        scratch_types=[
            pltpu.VMEM((GPW,), jnp.int32),
            pltpu.VMEM((GPW, H), jnp.float32),
            pltpu.SemaphoreType.DMA,
        ],
    )
    return f(hidden, all_idx)


def kernel(node_emb, type_emb, ln_in_g, ln_in_b,
           basis0, comp0, root0, bias0, ln0_g, ln0_b,
           basis1, comp1, root1, bias1, ln1_g, ln1_b,
           pw1, pb1, pw2, pb2, pw3, pb3,
           node_type_ids, edge_index, edge_type, drug_indices,
           disease_indices):
    edge_src = edge_index[0]
    edge_dst = edge_index[1]
    r2 = lambda a: a.reshape(1, -1)

    w, gidx = _sc_prep(edge_src, edge_dst, edge_type)

    def pad2(a, fill):
        return jnp.pad(a.reshape(NS, E // NS), ((0, 0), (0, EPT2 - E // NS)),
                       constant_values=fill).reshape(EPAD)
    w_p = pad2(w, 0.0)
    gidx_p = pad2(gidx, 0)
    dst_p = pad2(edge_dst, N)

    ids3 = node_type_ids.reshape(NB, 1, BN)
    hidden = _embed_ln(ids3, node_emb, type_emb, r2(ln_in_g), r2(ln_in_b))

    for basis, comp, root, bias, g, b in (
            (basis0, comp0, root0, bias0, ln0_g, ln0_b),
            (basis1, comp1, root1, bias1, ln1_g, ln1_b)):
        wflat = _wcomp(comp, basis.reshape(B_BASES, H * H))
        wstack = wflat.reshape(R, H, H)
        hlo, hhi = _hrel(hidden, wstack)
        agg2 = _sc_edge(hlo.reshape(R * N, HH), hhi.reshape(R * N, HH),
                        gidx_p.reshape(MROWS, C2), dst_p.reshape(MROWS, C2),
                        w_p.reshape(MROWS, C2)).reshape(2, N, HH)
        hidden = _update(hidden, agg2, root, r2(bias), r2(g), r2(b))

    all_idx = jnp.concatenate([drug_indices, disease_indices])
    pairs = _pair_gather(hidden, all_idx)
    logits = _mlp(pairs[:P], pairs[P:], pw1, r2(pb1), pw2, r2(pb2),
                  pw3, pb3.reshape(1, 1))
    return logits.reshape(P)
